# baseline (speedup 1.0000x reference)
import functools

import jax
import jax.numpy as jnp
from jax import lax
from jax.experimental import pallas as pl
from jax.experimental.pallas import tpu as pltpu
from jax.experimental.pallas import tpu_sc as plsc

N_NODES = 10000
N_EDGES = 160000
N_GRAPHS = 512
NPAD = 10240
DUMMY = N_NODES
NW = 32
EC = 128
E_PER_W = 5120
EPAD = NW * E_PER_W
ROWS_PER_S = NPAD // 16

_mesh = plsc.VectorSubcoreMesh(core_axis_name="c", subcore_axis_name="s")


def _sc_degree(dst_pad, zeros8, ones8):
    @functools.partial(
        pl.kernel, mesh=_mesh,
        compiler_params=pltpu.CompilerParams(use_tc_tiling_on_sc=False),
        out_type=jax.ShapeDtypeStruct((2, NPAD, 8), jnp.float32),
        scratch_types=[
            pltpu.VMEM((EC,), jnp.int32),
            pltpu.VMEM((EC, 8), jnp.float32),
            pltpu.VMEM_SHARED((NPAD, 8), jnp.float32),
            pltpu.SemaphoreType.DMA,
        ],
    )
    def k(dst_hbm, zeros_hbm, ones_hbm, out_hbm, idx_v, ones_v, acc, sem):
        c = lax.axis_index("c")
        s = lax.axis_index("s")
        wid = s * 2 + c
        pltpu.sync_copy(zeros_hbm.at[pl.ds(s * ROWS_PER_S, ROWS_PER_S)],
                        acc.at[pl.ds(s * ROWS_PER_S, ROWS_PER_S)])
        pltpu.sync_copy(ones_hbm, ones_v)
        plsc.subcore_barrier()
        base = wid * E_PER_W

        def body(i, carry):
            off = pl.multiple_of(base + i * EC, 8)
            pltpu.sync_copy(dst_hbm.at[pl.ds(off, EC)], idx_v)
            pltpu.sync_copy(ones_v, acc.at[idx_v], add=True)
            return carry

        lax.fori_loop(0, E_PER_W // EC, body, 0)
        plsc.subcore_barrier()
        pltpu.sync_copy(acc.at[pl.ds(s * ROWS_PER_S, ROWS_PER_S)],
                        out_hbm.at[c, pl.ds(s * ROWS_PER_S, ROWS_PER_S)])

    return k(dst_pad, zeros8, ones8)


def _sc_aggregate(src_pad, dst_pad, hs, zeros, d):
    @functools.partial(
        pl.kernel, mesh=_mesh,
        compiler_params=pltpu.CompilerParams(use_tc_tiling_on_sc=False),
        out_type=jax.ShapeDtypeStruct((2, NPAD, d), jnp.float32),
        scratch_types=[
            pltpu.VMEM((2, EC), jnp.int32),
            pltpu.VMEM((2, EC), jnp.int32),
            pltpu.VMEM((EC, d), jnp.float32),
            pltpu.VMEM((EC, d), jnp.float32),
            pltpu.VMEM_SHARED((NPAD, d), jnp.float32),
            pltpu.SemaphoreType.DMA,
            pltpu.SemaphoreType.DMA,
        ],
    )
    def k(src_hbm, dst_hbm, hs_hbm, zeros_hbm, out_hbm,
          src_v, dst_v, rows_v0, rows_v1, acc, sem0, sem1):
        c = lax.axis_index("c")
        s = lax.axis_index("s")
        wid = s * 2 + c
        pltpu.sync_copy(zeros_hbm.at[pl.ds(s * ROWS_PER_S, ROWS_PER_S)],
                        acc.at[pl.ds(s * ROWS_PER_S, ROWS_PER_S)])
        plsc.subcore_barrier()
        base = wid * (E_PER_W // EC)

        def body(i, carry):
            r = pl.multiple_of(base + i * 2, 2)
            pltpu.sync_copy(src_hbm.at[pl.ds(r, 2)], src_v)
            pltpu.sync_copy(dst_hbm.at[pl.ds(r, 2)], dst_v)
            cp0 = pltpu.async_copy(hs_hbm.at[src_v.at[0]], rows_v0, sem0)
            cp1 = pltpu.async_copy(hs_hbm.at[src_v.at[1]], rows_v1, sem1)
            cp0.wait()
            pltpu.sync_copy(rows_v0, acc.at[dst_v.at[0]], add=True)
            cp1.wait()
            pltpu.sync_copy(rows_v1, acc.at[dst_v.at[1]], add=True)
            return carry

        lax.fori_loop(0, E_PER_W // EC // 2, body, 0)
        plsc.subcore_barrier()
        pltpu.sync_copy(acc.at[pl.ds(s * ROWS_PER_S, ROWS_PER_S)],
                        out_hbm.at[c, pl.ds(s * ROWS_PER_S, ROWS_PER_S)])

    return k(src_pad, dst_pad, hs, zeros)


def _sc_rowgather(table, idx):
    b_per_w = N_GRAPHS // NW
    d = table.shape[1]

    @functools.partial(
        pl.kernel, mesh=_mesh,
        compiler_params=pltpu.CompilerParams(use_tc_tiling_on_sc=False),
        out_type=jax.ShapeDtypeStruct((N_GRAPHS, d), jnp.float32),
        scratch_types=[
            pltpu.VMEM((b_per_w,), jnp.int32),
            pltpu.VMEM((b_per_w, d), jnp.float32),
            pltpu.SemaphoreType.DMA,
        ],
    )
    def k(table_hbm, idx_hbm, out_hbm, idx_v, rows_v, sem):
        wid = lax.axis_index("s") * 2 + lax.axis_index("c")
        base = wid * b_per_w
        pltpu.sync_copy(idx_hbm.at[pl.ds(base, b_per_w)], idx_v)
        pltpu.async_copy(table_hbm.at[idx_v], rows_v, sem).wait()
        pltpu.sync_copy(rows_v, out_hbm.at[pl.ds(base, b_per_w)])

    return k(table, idx)


def _prep(degp, x_pad):
    dx = x_pad.shape[1]

    def body(degp_ref, x_ref, dinv_ref, xs_ref):
        deg = degp_ref[0] + degp_ref[1] + 1.0
        dinv = lax.rsqrt(deg)
        dinv_ref[...] = dinv
        xs_ref[...] = x_ref[...] * dinv[:, :1]

    return pl.pallas_call(
        body,
        out_shape=(jax.ShapeDtypeStruct((NPAD, 8), jnp.float32),
                   jax.ShapeDtypeStruct((NPAD, dx), jnp.float32)),
    )(degp, x_pad)


_RB = 2048


def _post(aggp, x, dinv8, wt, b8):
    din, dout = wt.shape

    def body(agg_ref, x_ref, dinv_ref, wt_ref, b_ref, h_ref, xs_ref):
        i = pl.program_id(0)
        dinv = dinv_ref[:, :1]
        y = dinv * (agg_ref[0] + agg_ref[1]) + (dinv * dinv) * x_ref[...]
        h = jnp.dot(y, wt_ref[...], preferred_element_type=jnp.float32)
        h = jnp.maximum(h + b_ref[:1, :], 0.0)
        rowid = i * _RB + lax.broadcasted_iota(jnp.int32, (_RB, 1), 0)
        h = jnp.where(rowid < N_NODES, h, 0.0)
        h_ref[...] = h
        xs_ref[...] = h * dinv

    return pl.pallas_call(
        body,
        grid=(NPAD // _RB,),
        in_specs=[
            pl.BlockSpec((2, _RB, din), lambda i: (0, i, 0)),
            pl.BlockSpec((_RB, din), lambda i: (i, 0)),
            pl.BlockSpec((_RB, 8), lambda i: (i, 0)),
            pl.BlockSpec((din, dout), lambda i: (0, 0)),
            pl.BlockSpec((8, dout), lambda i: (0, 0)),
        ],
        out_specs=(pl.BlockSpec((_RB, dout), lambda i: (i, 0)),
                   pl.BlockSpec((_RB, dout), lambda i: (i, 0))),
        out_shape=(jax.ShapeDtypeStruct((NPAD, dout), jnp.float32),
                   jax.ShapeDtypeStruct((NPAD, dout), jnp.float32)),
    )(aggp, x, dinv8, wt, b8)


def _segscan_max(h, seg8):
    d = h.shape[1]

    def body(h_ref, seg_ref, out_ref, cm_ref, cs_ref):
        i = pl.program_id(0)

        @pl.when(i == 0)
        def _():
            cs_ref[0, 0] = jnp.int32(-1)
            cm_ref[...] = jnp.zeros((8, d), jnp.float32)

        m = h_ref[...]
        seg = seg_ref[:, :1]
        k = 1
        while k < _RB:
            m_sh = jnp.concatenate(
                [jnp.zeros((k, d), jnp.float32), m[:-k, :]], axis=0)
            seg_sh = jnp.concatenate(
                [jnp.full((k, 1), -1, jnp.int32), seg[:-k, :]], axis=0)
            m = jnp.where(seg_sh == seg, jnp.maximum(m, m_sh), m)
            k *= 2
        m = jnp.where(seg == cs_ref[0, 0], jnp.maximum(m, cm_ref[:1, :]), m)
        out_ref[...] = m
        cm_ref[...] = jnp.broadcast_to(m[-1:, :], (8, d))
        cs_ref[0, 0] = seg_ref[_RB - 1, 0]

    return pl.pallas_call(
        body,
        grid=(NPAD // _RB,),
        in_specs=[
            pl.BlockSpec((_RB, d), lambda i: (i, 0)),
            pl.BlockSpec((_RB, 8), lambda i: (i, 0)),
        ],
        out_specs=pl.BlockSpec((_RB, d), lambda i: (i, 0)),
        out_shape=jax.ShapeDtypeStruct((NPAD, d), jnp.float32),
        scratch_shapes=[pltpu.VMEM((8, d), jnp.float32),
                        pltpu.SMEM((1, 1), jnp.int32)],
    )(h, seg8)


def _head(p1, p2, m1, m2, cell_pad, w):
    def body(p1_ref, p2_ref, m1_ref, m2_ref, cell_ref,
             g1wt, g1b, g2wt, g2b, r1wt, r1b, r2wt, r2b, r3wt, r3b,
             sgwt, sgb, f1wt, f1b, f2wt, f2b, owt, ob, out_ref):
        def dot(a, b):
            return jnp.dot(a, b[...], preferred_element_type=jnp.float32)

        def enc(p_ref, m_ref):
            p = p_ref[...] * m_ref[:, :1]
            g = jnp.maximum(dot(p, g1wt) + g1b[:1, :], 0.0)
            return dot(g, g2wt) + g2b[:1, :]

        d1 = enc(p1_ref, m1_ref)
        d2 = enc(p2_ref, m2_ref)
        cell = cell_ref[...]
        cn = jnp.sqrt(jnp.sum(cell * cell, axis=1, keepdims=True))
        cv = cell / jnp.maximum(cn, 1e-12)
        cv = jnp.maximum(dot(cv, r1wt) + r1b[:1, :], 0.0)
        cv = jnp.maximum(dot(cv, r2wt) + r2b[:1, :], 0.0)
        cv = dot(cv, r3wt) + r3b[:1, :]
        dd = jnp.concatenate([d1, d2], axis=1)
        gate = jax.nn.sigmoid(dot(dd, sgwt) + sgb[:1, :])
        xc = jnp.concatenate([dd, gate, cv], axis=1)
        xc = jnp.maximum(dot(xc, f1wt) + f1b[:1, :], 0.0)
        xc = jnp.maximum(dot(xc, f2wt) + f2b[:1, :], 0.0)
        out_ref[...] = dot(xc, owt) + ob[:1, :]

    return pl.pallas_call(
        body,
        out_shape=jax.ShapeDtypeStruct((N_GRAPHS, 128), jnp.float32),
    )(p1, p2, m1, m2, cell_pad, *w)


def _pad_wt(wmat, rpad, cpad):
    wt = wmat.T
    return jnp.zeros((rpad, cpad), jnp.float32).at[
        :wt.shape[0], :wt.shape[1]].set(wt)


def _pad_b(b, cpad):
    row = jnp.zeros((cpad,), jnp.float32).at[:b.shape[0]].set(b)
    return jnp.broadcast_to(row[None, :], (8, cpad))


def kernel(x1, edge_index1, batch1, x2, edge_index2, batch2, cell, params):
    p = params
    f32 = jnp.float32

    w1t = _pad_wt(p['c1w'], 80, 80)
    b1 = _pad_b(p['c1b'], 80)
    w2t = _pad_wt(p['c2w'], 80, 160)
    b2 = _pad_b(p['c2b'], 160)
    w3t = _pad_wt(p['c3w'], 160, 320)
    b3 = _pad_b(p['c3b'], 320)

    zeros8 = jnp.zeros((NPAD, 8), f32)
    ones8 = jnp.ones((EC, 8), f32)
    zeros80 = jnp.zeros((NPAD, 80), f32)
    epad_fill = jnp.full((EPAD - N_EDGES,), DUMMY, jnp.int32)

    def encode(x, ei, batch):
        src = jnp.concatenate([ei[0].astype(jnp.int32), epad_fill])
        dst = jnp.concatenate([ei[1].astype(jnp.int32), epad_fill])
        x_pad = jnp.zeros((NPAD, 80), f32).at[:N_NODES, :78].set(x)

        degp = _sc_degree(dst, zeros8, ones8)
        dinv8, xs = _prep(degp, x_pad)

        src2 = src.reshape(EPAD // EC, EC)
        dst2 = dst.reshape(EPAD // EC, EC)
        agg = _sc_aggregate(src2, dst2, xs, zeros80, 80)
        h, xs = _post(agg, x_pad, dinv8, w1t, b1)
        agg = _sc_aggregate(src2, dst2, xs, zeros80, 80)
        h, xs = _post(agg, h, dinv8, w2t, b2)
        agg_a = _sc_aggregate(src2, dst2, xs[:, :80], zeros80, 80)
        agg_b = _sc_aggregate(src2, dst2, xs[:, 80:], zeros80, 80)
        agg = jnp.concatenate([agg_a, agg_b], axis=2)
        h3, _ = _post(agg, h, dinv8, w3t, b3)

        batch32 = batch.astype(jnp.int32)
        seg8 = jnp.broadcast_to(
            jnp.concatenate([batch32, jnp.full((NPAD - N_NODES,),
                                               N_GRAPHS - 1, jnp.int32)]
                            )[:, None], (NPAD, 8))
        h_scan = _segscan_max(h3, seg8)

        gids = jnp.arange(N_GRAPHS, dtype=jnp.int32)
        end = jnp.searchsorted(batch32, gids, side='right').astype(jnp.int32)
        start = jnp.searchsorted(batch32, gids, side='left').astype(jnp.int32)
        nonempty = (end > start) | (gids == N_GRAPHS - 1)
        last = jnp.where(gids == N_GRAPHS - 1, NPAD - 1, end - 1)
        last = jnp.maximum(last, 0)
        pool = _sc_rowgather(h_scan, last)
        mask = jnp.broadcast_to(
            nonempty.astype(f32)[:, None], (N_GRAPHS, 8))
        return pool, mask

    pool1, m1 = encode(x1, edge_index1, batch1)
    pool2, m2 = encode(x2, edge_index2, batch2)

    cell_pad = jnp.zeros((N_GRAPHS, 960), f32).at[:, :954].set(cell)
    w = (
        _pad_wt(p['g1w'], 320, 160), _pad_b(p['g1b'], 160),
        _pad_wt(p['g2w'], 160, 128), _pad_b(p['g2b'], 128),
        _pad_wt(p['r1w'], 960, 512), _pad_b(p['r1b'], 512),
        _pad_wt(p['r2w'], 512, 256), _pad_b(p['r2b'], 256),
        _pad_wt(p['r3w'], 256, 128), _pad_b(p['r3b'], 128),
        _pad_wt(p['sgw'], 256, 128), _pad_b(p['sgb'], 128),
        _pad_wt(p['f1w'], 512, 512), _pad_b(p['f1b'], 512),
        _pad_wt(p['f2w'], 512, 128), _pad_b(p['f2b'], 128),
        _pad_wt(p['ow'], 128, 128), _pad_b(p['ob'], 128),
    )
    out = _head(pool1, pool2, m1, m2, cell_pad, w)
    return out[:, :2]


# speedup vs baseline: 1.1272x; 1.1272x over previous
import functools

import jax
import jax.numpy as jnp
from jax import lax
from jax.experimental import pallas as pl
from jax.experimental.pallas import tpu as pltpu
from jax.experimental.pallas import tpu_sc as plsc

N_NODES = 10000
N_EDGES = 160000
N_GRAPHS = 512
NPAD = 10240
DUMMY = N_NODES
NW = 32
EC = 128
E_PER_W = 5120
EPAD = NW * E_PER_W
ROWS_PER_S = NPAD // 16

_mesh = plsc.VectorSubcoreMesh(core_axis_name="c", subcore_axis_name="s")


def _sc_degree(dst_pad, zeros8, ones8):
    @functools.partial(
        pl.kernel, mesh=_mesh,
        compiler_params=pltpu.CompilerParams(use_tc_tiling_on_sc=False),
        out_type=jax.ShapeDtypeStruct((2, NPAD, 8), jnp.float32),
        scratch_types=[
            pltpu.VMEM((EC,), jnp.int32),
            pltpu.VMEM((EC, 8), jnp.float32),
            pltpu.VMEM_SHARED((NPAD, 8), jnp.float32),
            pltpu.SemaphoreType.DMA,
        ],
    )
    def k(dst_hbm, zeros_hbm, ones_hbm, out_hbm, idx_v, ones_v, acc, sem):
        c = lax.axis_index("c")
        s = lax.axis_index("s")
        wid = s * 2 + c
        pltpu.sync_copy(zeros_hbm.at[pl.ds(s * ROWS_PER_S, ROWS_PER_S)],
                        acc.at[pl.ds(s * ROWS_PER_S, ROWS_PER_S)])
        pltpu.sync_copy(ones_hbm, ones_v)
        plsc.subcore_barrier()
        base = wid * E_PER_W

        def body(i, carry):
            off = pl.multiple_of(base + i * EC, 8)
            pltpu.sync_copy(dst_hbm.at[pl.ds(off, EC)], idx_v)
            pltpu.sync_copy(ones_v, acc.at[idx_v], add=True)
            return carry

        lax.fori_loop(0, E_PER_W // EC, body, 0)
        plsc.subcore_barrier()
        pltpu.sync_copy(acc.at[pl.ds(s * ROWS_PER_S, ROWS_PER_S)],
                        out_hbm.at[c, pl.ds(s * ROWS_PER_S, ROWS_PER_S)])

    return k(dst_pad, zeros8, ones8)


def _sc_aggregate(src_pad, dst_pad, hs, zeros, d, db=True):
    nbuf = 2 if db else 1

    @functools.partial(
        pl.kernel, mesh=_mesh,
        compiler_params=pltpu.CompilerParams(use_tc_tiling_on_sc=False),
        out_type=jax.ShapeDtypeStruct((2, NPAD, d), jnp.float32),
        scratch_types=[
            pltpu.VMEM((nbuf, EC), jnp.int32),
            pltpu.VMEM((nbuf, EC), jnp.int32),
            [pltpu.VMEM((EC, d), jnp.float32) for _ in range(nbuf)],
            pltpu.VMEM_SHARED((NPAD, d), jnp.float32),
            [pltpu.SemaphoreType.DMA for _ in range(nbuf)],
        ],
    )
    def k(src_hbm, dst_hbm, hs_hbm, zeros_hbm, out_hbm,
          src_v, dst_v, rows_v, acc, sems):
        c = lax.axis_index("c")
        s = lax.axis_index("s")
        wid = s * 2 + c
        pltpu.sync_copy(zeros_hbm.at[pl.ds(s * ROWS_PER_S, ROWS_PER_S)],
                        acc.at[pl.ds(s * ROWS_PER_S, ROWS_PER_S)])
        plsc.subcore_barrier()
        base = wid * (E_PER_W // EC)

        def body(i, carry):
            r = pl.multiple_of(base + i * nbuf, 1)
            pltpu.sync_copy(src_hbm.at[pl.ds(r, nbuf)], src_v)
            pltpu.sync_copy(dst_hbm.at[pl.ds(r, nbuf)], dst_v)
            cps = [pltpu.async_copy(hs_hbm.at[src_v.at[b]], rows_v[b],
                                    sems[b]) for b in range(nbuf)]
            for b in range(nbuf):
                cps[b].wait()
                pltpu.sync_copy(rows_v[b], acc.at[dst_v.at[b]], add=True)
            return carry

        lax.fori_loop(0, E_PER_W // EC // nbuf, body, 0)
        plsc.subcore_barrier()
        pltpu.sync_copy(acc.at[pl.ds(s * ROWS_PER_S, ROWS_PER_S)],
                        out_hbm.at[c, pl.ds(s * ROWS_PER_S, ROWS_PER_S)])

    return k(src_pad, dst_pad, hs, zeros)


def _sc_rowgather(table, idx):
    b_per_w = N_GRAPHS // NW
    d = table.shape[1]

    @functools.partial(
        pl.kernel, mesh=_mesh,
        compiler_params=pltpu.CompilerParams(use_tc_tiling_on_sc=False),
        out_type=jax.ShapeDtypeStruct((N_GRAPHS, d), jnp.float32),
        scratch_types=[
            pltpu.VMEM((b_per_w,), jnp.int32),
            pltpu.VMEM((b_per_w, d), jnp.float32),
            pltpu.SemaphoreType.DMA,
        ],
    )
    def k(table_hbm, idx_hbm, out_hbm, idx_v, rows_v, sem):
        wid = lax.axis_index("s") * 2 + lax.axis_index("c")
        base = wid * b_per_w
        pltpu.sync_copy(idx_hbm.at[pl.ds(base, b_per_w)], idx_v)
        pltpu.async_copy(table_hbm.at[idx_v], rows_v, sem).wait()
        pltpu.sync_copy(rows_v, out_hbm.at[pl.ds(base, b_per_w)])

    return k(table, idx)


def _prep(degp, x_pad):
    dx = x_pad.shape[1]

    def body(degp_ref, x_ref, dinv_ref, xs_ref):
        deg = degp_ref[0] + degp_ref[1] + 1.0
        dinv = lax.rsqrt(deg)
        dinv_ref[...] = dinv
        xs_ref[...] = x_ref[...] * dinv[:, :1]

    return pl.pallas_call(
        body,
        out_shape=(jax.ShapeDtypeStruct((NPAD, 8), jnp.float32),
                   jax.ShapeDtypeStruct((NPAD, dx), jnp.float32)),
    )(degp, x_pad)


_RB = 2048


def _post(aggp, x, dinv8, wt, b8):
    din, dout = wt.shape

    def body(agg_ref, x_ref, dinv_ref, wt_ref, b_ref, h_ref, xs_ref):
        i = pl.program_id(0)
        dinv = dinv_ref[:, :1]
        y = dinv * (agg_ref[0] + agg_ref[1]) + (dinv * dinv) * x_ref[...]
        h = jnp.dot(y, wt_ref[...], preferred_element_type=jnp.float32)
        h = jnp.maximum(h + b_ref[:1, :], 0.0)
        rowid = i * _RB + lax.broadcasted_iota(jnp.int32, (_RB, 1), 0)
        h = jnp.where(rowid < N_NODES, h, 0.0)
        h_ref[...] = h
        xs_ref[...] = h * dinv

    return pl.pallas_call(
        body,
        grid=(NPAD // _RB,),
        in_specs=[
            pl.BlockSpec((2, _RB, din), lambda i: (0, i, 0)),
            pl.BlockSpec((_RB, din), lambda i: (i, 0)),
            pl.BlockSpec((_RB, 8), lambda i: (i, 0)),
            pl.BlockSpec((din, dout), lambda i: (0, 0)),
            pl.BlockSpec((8, dout), lambda i: (0, 0)),
        ],
        out_specs=(pl.BlockSpec((_RB, dout), lambda i: (i, 0)),
                   pl.BlockSpec((_RB, dout), lambda i: (i, 0))),
        out_shape=(jax.ShapeDtypeStruct((NPAD, dout), jnp.float32),
                   jax.ShapeDtypeStruct((NPAD, dout), jnp.float32)),
    )(aggp, x, dinv8, wt, b8)


def _segscan_max(h, seg8):
    d = h.shape[1]

    def body(h_ref, seg_ref, out_ref, cm_ref, cs_ref):
        i = pl.program_id(0)

        @pl.when(i == 0)
        def _():
            cs_ref[0, 0] = jnp.int32(-1)
            cm_ref[...] = jnp.zeros((8, d), jnp.float32)

        m = h_ref[...]
        seg = seg_ref[:, :1]
        k = 1
        while k < _RB:
            m_sh = jnp.concatenate(
                [jnp.zeros((k, d), jnp.float32), m[:-k, :]], axis=0)
            seg_sh = jnp.concatenate(
                [jnp.full((k, 1), -1, jnp.int32), seg[:-k, :]], axis=0)
            m = jnp.where(seg_sh == seg, jnp.maximum(m, m_sh), m)
            k *= 2
        m = jnp.where(seg == cs_ref[0, 0], jnp.maximum(m, cm_ref[:1, :]), m)
        out_ref[...] = m
        cm_ref[...] = jnp.broadcast_to(m[-1:, :], (8, d))
        cs_ref[0, 0] = seg_ref[_RB - 1, 0]

    return pl.pallas_call(
        body,
        grid=(NPAD // _RB,),
        in_specs=[
            pl.BlockSpec((_RB, d), lambda i: (i, 0)),
            pl.BlockSpec((_RB, 8), lambda i: (i, 0)),
        ],
        out_specs=pl.BlockSpec((_RB, d), lambda i: (i, 0)),
        out_shape=jax.ShapeDtypeStruct((NPAD, d), jnp.float32),
        scratch_shapes=[pltpu.VMEM((8, d), jnp.float32),
                        pltpu.SMEM((1, 1), jnp.int32)],
    )(h, seg8)


def _head(p1, p2, m1, m2, cell_pad, w):
    def body(p1_ref, p2_ref, m1_ref, m2_ref, cell_ref,
             g1wt, g1b, g2wt, g2b, r1wt, r1b, r2wt, r2b, r3wt, r3b,
             sgwt, sgb, f1wt, f1b, f2wt, f2b, owt, ob, out_ref):
        def dot(a, b):
            return jnp.dot(a, b[...], preferred_element_type=jnp.float32)

        def enc(p_ref, m_ref):
            p = p_ref[...] * m_ref[:, :1]
            g = jnp.maximum(dot(p, g1wt) + g1b[:1, :], 0.0)
            return dot(g, g2wt) + g2b[:1, :]

        d1 = enc(p1_ref, m1_ref)
        d2 = enc(p2_ref, m2_ref)
        cell = cell_ref[...]
        cn = jnp.sqrt(jnp.sum(cell * cell, axis=1, keepdims=True))
        cv = cell / jnp.maximum(cn, 1e-12)
        cv = jnp.maximum(dot(cv, r1wt) + r1b[:1, :], 0.0)
        cv = jnp.maximum(dot(cv, r2wt) + r2b[:1, :], 0.0)
        cv = dot(cv, r3wt) + r3b[:1, :]
        dd = jnp.concatenate([d1, d2], axis=1)
        gate = jax.nn.sigmoid(dot(dd, sgwt) + sgb[:1, :])
        xc = jnp.concatenate([dd, gate, cv], axis=1)
        xc = jnp.maximum(dot(xc, f1wt) + f1b[:1, :], 0.0)
        xc = jnp.maximum(dot(xc, f2wt) + f2b[:1, :], 0.0)
        out_ref[...] = dot(xc, owt) + ob[:1, :]

    return pl.pallas_call(
        body,
        out_shape=jax.ShapeDtypeStruct((N_GRAPHS, 128), jnp.float32),
    )(p1, p2, m1, m2, cell_pad, *w)


def _pad_wt(wmat, rpad, cpad):
    wt = wmat.T
    return jnp.zeros((rpad, cpad), jnp.float32).at[
        :wt.shape[0], :wt.shape[1]].set(wt)


def _pad_b(b, cpad):
    row = jnp.zeros((cpad,), jnp.float32).at[:b.shape[0]].set(b)
    return jnp.broadcast_to(row[None, :], (8, cpad))


def kernel(x1, edge_index1, batch1, x2, edge_index2, batch2, cell, params):
    p = params
    f32 = jnp.float32

    w1t = _pad_wt(p['c1w'], 80, 80)
    b1 = _pad_b(p['c1b'], 80)
    w2t = _pad_wt(p['c2w'], 80, 160)
    b2 = _pad_b(p['c2b'], 160)
    w3t = _pad_wt(p['c3w'], 160, 320)
    b3 = _pad_b(p['c3b'], 320)

    zeros8 = jnp.zeros((NPAD, 8), f32)
    ones8 = jnp.ones((EC, 8), f32)
    zeros80 = jnp.zeros((NPAD, 80), f32)
    zeros160 = jnp.zeros((NPAD, 160), f32)
    epad_fill = jnp.full((EPAD - N_EDGES,), DUMMY, jnp.int32)

    def encode(x, ei, batch):
        src = jnp.concatenate([ei[0].astype(jnp.int32), epad_fill])
        dst = jnp.concatenate([ei[1].astype(jnp.int32), epad_fill])
        x_pad = jnp.zeros((NPAD, 80), f32).at[:N_NODES, :78].set(x)

        degp = _sc_degree(dst, zeros8, ones8)
        dinv8, xs = _prep(degp, x_pad)

        src2 = src.reshape(EPAD // EC, EC)
        dst2 = dst.reshape(EPAD // EC, EC)
        agg = _sc_aggregate(src2, dst2, xs, zeros80, 80)
        h, xs = _post(agg, x_pad, dinv8, w1t, b1)
        agg = _sc_aggregate(src2, dst2, xs, zeros80, 80)
        h, xs = _post(agg, h, dinv8, w2t, b2)
        agg = _sc_aggregate(src2, dst2, xs, zeros160, 160, db=False)
        h3, _ = _post(agg, h, dinv8, w3t, b3)

        batch32 = batch.astype(jnp.int32)
        seg8 = jnp.broadcast_to(
            jnp.concatenate([batch32, jnp.full((NPAD - N_NODES,),
                                               N_GRAPHS - 1, jnp.int32)]
                            )[:, None], (NPAD, 8))
        h_scan = _segscan_max(h3, seg8)

        gids = jnp.arange(N_GRAPHS, dtype=jnp.int32)
        end = jnp.searchsorted(batch32, gids, side='right').astype(jnp.int32)
        start = jnp.searchsorted(batch32, gids, side='left').astype(jnp.int32)
        nonempty = (end > start) | (gids == N_GRAPHS - 1)
        last = jnp.where(gids == N_GRAPHS - 1, NPAD - 1, end - 1)
        last = jnp.maximum(last, 0)
        pool = _sc_rowgather(h_scan, last)
        mask = jnp.broadcast_to(
            nonempty.astype(f32)[:, None], (N_GRAPHS, 8))
        return pool, mask

    pool1, m1 = encode(x1, edge_index1, batch1)
    pool2, m2 = encode(x2, edge_index2, batch2)

    cell_pad = jnp.zeros((N_GRAPHS, 960), f32).at[:, :954].set(cell)
    w = (
        _pad_wt(p['g1w'], 320, 160), _pad_b(p['g1b'], 160),
        _pad_wt(p['g2w'], 160, 128), _pad_b(p['g2b'], 128),
        _pad_wt(p['r1w'], 960, 512), _pad_b(p['r1b'], 512),
        _pad_wt(p['r2w'], 512, 256), _pad_b(p['r2b'], 256),
        _pad_wt(p['r3w'], 256, 128), _pad_b(p['r3b'], 128),
        _pad_wt(p['sgw'], 256, 128), _pad_b(p['sgb'], 128),
        _pad_wt(p['f1w'], 512, 512), _pad_b(p['f1b'], 512),
        _pad_wt(p['f2w'], 512, 128), _pad_b(p['f2b'], 128),
        _pad_wt(p['ow'], 128, 128), _pad_b(p['ob'], 128),
    )
    out = _head(pool1, pool2, m1, m2, cell_pad, w)
    return out[:, :2]
